# Initial kernel scaffold; baseline (speedup 1.0000x reference)
import jax
import jax.numpy as jnp
from jax.experimental import pallas as pl


def kernel(x, hidden_states, masks, W_ih_0, W_hh_0, b_ih_0, b_hh_0, W_ih_1, W_hh_1, b_ih_1, b_hh_1):
    raise NotImplementedError("write your pallas kernel here")


# speedup vs baseline: 3.2270x; 3.2270x over previous
import jax
import jax.numpy as jnp
from jax.experimental import pallas as pl
from jax.experimental.pallas import tpu as pltpu


def _matmul_bias_kernel(x_ref, w_ref, b_ref, o_ref):
    o_ref[...] = (
        jnp.dot(x_ref[...], w_ref[...], preferred_element_type=jnp.float32)
        + b_ref[...]
    )


def _matmul_bias(x, wt, b, bm=512):
    m, k = x.shape
    _, n = wt.shape
    return pl.pallas_call(
        _matmul_bias_kernel,
        grid=(m // bm,),
        in_specs=[
            pl.BlockSpec((bm, k), lambda i: (i, 0)),
            pl.BlockSpec((k, n), lambda i: (0, 0)),
            pl.BlockSpec((1, n), lambda i: (0, 0)),
        ],
        out_specs=pl.BlockSpec((bm, n), lambda i: (i, 0)),
        out_shape=jax.ShapeDtypeStruct((m, n), jnp.float32),
    )(x, wt, b)


def _gru_scan_kernel(gi_ref, m_ref, h0_ref, whh_ref, bhh_ref,
                     out_ref, hn_ref, h_scr):
    t = pl.program_id(0)
    nsteps = pl.num_programs(0)
    hdim = h0_ref.shape[-1]

    @pl.when(t == 0)
    def _():
        h_scr[...] = h0_ref[...]

    h = h_scr[...]
    m_t = m_ref[0]
    hm = h * m_t
    gi = gi_ref[0]
    gh = (
        jnp.dot(hm, whh_ref[...], preferred_element_type=jnp.float32)
        + bhh_ref[...]
    )
    r = jax.nn.sigmoid(gi[:, :hdim] + gh[:, :hdim])
    z = jax.nn.sigmoid(gi[:, hdim:2 * hdim] + gh[:, hdim:2 * hdim])
    n = jnp.tanh(gi[:, 2 * hdim:] + r * gh[:, 2 * hdim:])
    h_new = (1.0 - z) * n + z * hm

    out_ref[0] = h_new
    h_scr[...] = h_new

    @pl.when(t == nsteps - 1)
    def _():
        hn_ref[...] = h_new


def _gru_scan(gi, m3, h0, whh_t, bhh):
    t, n, g3 = gi.shape
    h = h0.shape[-1]
    return pl.pallas_call(
        _gru_scan_kernel,
        grid=(t,),
        in_specs=[
            pl.BlockSpec((1, n, g3), lambda i: (i, 0, 0)),
            pl.BlockSpec((1, n, 1), lambda i: (i, 0, 0)),
            pl.BlockSpec((n, h), lambda i: (0, 0)),
            pl.BlockSpec((h, g3), lambda i: (0, 0)),
            pl.BlockSpec((1, g3), lambda i: (0, 0)),
        ],
        out_specs=[
            pl.BlockSpec((1, n, h), lambda i: (i, 0, 0)),
            pl.BlockSpec((n, h), lambda i: (0, 0)),
        ],
        out_shape=[
            jax.ShapeDtypeStruct((t, n, h), jnp.float32),
            jax.ShapeDtypeStruct((n, h), jnp.float32),
        ],
        scratch_shapes=[pltpu.VMEM((n, h), jnp.float32)],
        compiler_params=pltpu.CompilerParams(
            dimension_semantics=("arbitrary",),
        ),
    )(gi, m3, h0, whh_t, bhh)


def kernel(x, hidden_states, masks, W_ih_0, W_hh_0, b_ih_0, b_hh_0,
           W_ih_1, W_hh_1, b_ih_1, b_hh_1):
    n = hidden_states.shape[1]
    tn, d = x.shape
    t = tn // n
    h = hidden_states.shape[2]

    m3 = masks.reshape(t, n, 1).astype(jnp.float32)

    gi0 = _matmul_bias(x, W_ih_0.T, b_ih_0.reshape(1, -1)).reshape(t, n, 3 * h)
    out0, h0 = _gru_scan(gi0, m3, hidden_states[0], W_hh_0.T,
                         b_hh_0.reshape(1, -1))

    gi1 = _matmul_bias(out0.reshape(tn, h), W_ih_1.T,
                       b_ih_1.reshape(1, -1)).reshape(t, n, 3 * h)
    out1, h1 = _gru_scan(gi1, m3, hidden_states[1], W_hh_1.T,
                         b_hh_1.reshape(1, -1))

    return out1.reshape(tn, h), jnp.stack([h0, h1], axis=0)


# speedup vs baseline: 6.4047x; 1.9847x over previous
import jax
import jax.numpy as jnp
from jax.experimental import pallas as pl
from jax.experimental.pallas import tpu as pltpu

_CHUNK = 16


def _gru_steps(chunk, hdim, h, m_ref, gi_scr, whh_ref, bhh_ref, out_wr):
    for i in range(chunk):
        m_t = m_ref[i]
        hm = h * m_t
        gi_t = gi_scr[i * 16:(i + 1) * 16, :]
        gh_r = jnp.dot(hm, whh_ref[:, :hdim],
                       preferred_element_type=jnp.float32) + bhh_ref[:, :hdim]
        gh_z = jnp.dot(hm, whh_ref[:, hdim:2 * hdim],
                       preferred_element_type=jnp.float32) + bhh_ref[:, hdim:2 * hdim]
        gh_n = jnp.dot(hm, whh_ref[:, 2 * hdim:],
                       preferred_element_type=jnp.float32) + bhh_ref[:, 2 * hdim:]
        r = jax.nn.sigmoid(gi_t[:, :hdim] + gh_r)
        z = jax.nn.sigmoid(gi_t[:, hdim:2 * hdim] + gh_z)
        n = jnp.tanh(gi_t[:, 2 * hdim:] + r * gh_n)
        h = (1.0 - z) * n + z * hm
        out_wr(i, h)
    return h


def _fused_kernel(x_ref, m_ref, h0_ref, h1_ref,
                  wih0_ref, bih0_ref, whh0_ref, bhh0_ref,
                  wih1_ref, bih1_ref, whh1_ref, bhh1_ref,
                  out_ref, h0n_ref, h1n_ref,
                  h0_scr, h1_scr, gi_scr, out0_scr):
    c = pl.program_id(0)
    nchunks = pl.num_programs(0)
    hdim = h0_ref.shape[-1]
    chunk = m_ref.shape[0]

    @pl.when(c == 0)
    def _():
        h0_scr[...] = h0_ref[...]
        h1_scr[...] = h1_ref[...]

    gi_scr[...] = jnp.dot(x_ref[...], wih0_ref[...],
                          preferred_element_type=jnp.float32) + bih0_ref[...]

    def wr0(i, h):
        out0_scr[i * 16:(i + 1) * 16, :] = h

    h0 = _gru_steps(chunk, hdim, h0_scr[...], m_ref, gi_scr,
                    whh0_ref, bhh0_ref, wr0)
    h0_scr[...] = h0

    gi_scr[...] = jnp.dot(out0_scr[...], wih1_ref[...],
                          preferred_element_type=jnp.float32) + bih1_ref[...]

    def wr1(i, h):
        out_ref[i * 16:(i + 1) * 16, :] = h

    h1 = _gru_steps(chunk, hdim, h1_scr[...], m_ref, gi_scr,
                    whh1_ref, bhh1_ref, wr1)
    h1_scr[...] = h1

    @pl.when(c == nchunks - 1)
    def _():
        h0n_ref[...] = h0
        h1n_ref[...] = h1


def kernel(x, hidden_states, masks, W_ih_0, W_hh_0, b_ih_0, b_hh_0,
           W_ih_1, W_hh_1, b_ih_1, b_hh_1):
    n = hidden_states.shape[1]
    tn, d = x.shape
    t = tn // n
    h = hidden_states.shape[2]
    g3 = 3 * h
    chunk = _CHUNK
    rows = chunk * n

    m3 = masks.reshape(t, n, 1).astype(jnp.float32)

    out, h0n, h1n = pl.pallas_call(
        _fused_kernel,
        grid=(t // chunk,),
        in_specs=[
            pl.BlockSpec((rows, d), lambda c: (c, 0)),
            pl.BlockSpec((chunk, n, 1), lambda c: (c, 0, 0)),
            pl.BlockSpec((n, h), lambda c: (0, 0)),
            pl.BlockSpec((n, h), lambda c: (0, 0)),
            pl.BlockSpec((d, g3), lambda c: (0, 0)),
            pl.BlockSpec((1, g3), lambda c: (0, 0)),
            pl.BlockSpec((h, g3), lambda c: (0, 0)),
            pl.BlockSpec((1, g3), lambda c: (0, 0)),
            pl.BlockSpec((h, g3), lambda c: (0, 0)),
            pl.BlockSpec((1, g3), lambda c: (0, 0)),
            pl.BlockSpec((h, g3), lambda c: (0, 0)),
            pl.BlockSpec((1, g3), lambda c: (0, 0)),
        ],
        out_specs=[
            pl.BlockSpec((rows, h), lambda c: (c, 0)),
            pl.BlockSpec((n, h), lambda c: (0, 0)),
            pl.BlockSpec((n, h), lambda c: (0, 0)),
        ],
        out_shape=[
            jax.ShapeDtypeStruct((tn, h), jnp.float32),
            jax.ShapeDtypeStruct((n, h), jnp.float32),
            jax.ShapeDtypeStruct((n, h), jnp.float32),
        ],
        scratch_shapes=[
            pltpu.VMEM((n, h), jnp.float32),
            pltpu.VMEM((n, h), jnp.float32),
            pltpu.VMEM((rows, g3), jnp.float32),
            pltpu.VMEM((rows, h), jnp.float32),
        ],
        compiler_params=pltpu.CompilerParams(
            dimension_semantics=("arbitrary",),
        ),
    )(x, m3, hidden_states[0], hidden_states[1],
      W_ih_0.T, b_ih_0.reshape(1, -1), W_hh_0.T, b_hh_0.reshape(1, -1),
      W_ih_1.T, b_ih_1.reshape(1, -1), W_hh_1.T, b_hh_1.reshape(1, -1))

    return out, jnp.stack([h0n, h1n], axis=0)
